# baseline (device time: 687953 ns/iter reference)
import jax
import jax.numpy as jnp
from jax import lax
from jax.experimental import pallas as pl
from jax.experimental.pallas import tpu as pltpu

NC = 16
EX = 5
NY = NC + EX
RX_LO, RX_HI = EX, 10
RZ_LO, RZ_HI = 10, NC
S_Y = 6
S_L = 4
S_SND = 4


def kernel(x):
    m, n = x.shape
    q_m = m // 4
    c_m = q_m // NC

    def body(x_ref, out_ref, recv_y, xloc,
             y_send, y_recv, load_sems, oc_sems, sx_send, sz_send,
             rxz_send, rzx_send, xin_sems, zin_sems, xr_sems, zr_sems):
        my_x = lax.axis_index("x")
        my_y = lax.axis_index("y")
        my_z = lax.axis_index("z")
        zbit = lax.rem(my_z, 2)
        ypeer = (my_x, 1 - my_y, my_z)
        xpeer = (1 - my_x, my_y, my_z)
        zpart = (my_x, my_y, my_z + 1 - 2 * zbit)

        o = 2 * my_x + zbit
        ox = lax.rem(o + 2, 4)
        oz = o + 1 - 2 * lax.rem(o, 2)
        od = lax.rem(oz + 2, 4)

        def rows(quarter, k):
            return pl.ds(quarter * q_m + k * c_m, c_m)

        def my_rows(k):
            if k < NC:
                return rows(o, k)
            return rows(od, k - NC)

        def y_rdma(k):
            return pltpu.make_async_remote_copy(
                src_ref=x_ref.at[my_rows(k), :],
                dst_ref=recv_y.at[k % S_Y],
                send_sem=y_send.at[k % S_SND],
                recv_sem=y_recv.at[k % S_Y],
                device_id=ypeer,
                device_id_type=pl.DeviceIdType.MESH,
            )

        def xload(k):
            return pltpu.make_async_copy(
                x_ref.at[my_rows(k), :],
                xloc.at[k % S_L],
                load_sems.at[k % S_L],
            )

        def out_copy(k):
            return pltpu.make_async_copy(
                xloc.at[k % S_L],
                out_ref.at[my_rows(k), :],
                oc_sems.at[k % S_SND],
            )

        def sum_x(k):
            return pltpu.make_async_remote_copy(
                src_ref=xloc.at[k % S_L],
                dst_ref=out_ref.at[rows(o, k), :],
                send_sem=sx_send.at[k % S_SND],
                recv_sem=xin_sems.at[k],
                device_id=xpeer,
                device_id_type=pl.DeviceIdType.MESH,
            )

        def sum_z(k):
            return pltpu.make_async_remote_copy(
                src_ref=xloc.at[k % S_L],
                dst_ref=out_ref.at[rows(o, k), :],
                send_sem=sz_send.at[k % S_SND],
                recv_sem=zin_sems.at[k],
                device_id=zpart,
                device_id_type=pl.DeviceIdType.MESH,
            )

        def xin(k):
            return pltpu.make_async_remote_copy(
                src_ref=xloc.at[0],
                dst_ref=out_ref.at[rows(ox, k), :],
                send_sem=sx_send.at[0],
                recv_sem=xin_sems.at[k],
                device_id=xpeer,
                device_id_type=pl.DeviceIdType.MESH,
            )

        def zin(k):
            return pltpu.make_async_remote_copy(
                src_ref=xloc.at[0],
                dst_ref=out_ref.at[rows(oz, k), :],
                send_sem=sz_send.at[0],
                recv_sem=zin_sems.at[k],
                device_id=zpart,
                device_id_type=pl.DeviceIdType.MESH,
            )

        def relay_xz(k):
            return pltpu.make_async_remote_copy(
                src_ref=out_ref.at[rows(ox, k), :],
                dst_ref=out_ref.at[rows(ox, k), :],
                send_sem=rxz_send.at[k - RZ_LO],
                recv_sem=zr_sems.at[k - RZ_LO],
                device_id=zpart,
                device_id_type=pl.DeviceIdType.MESH,
            )

        def relay_zx(k):
            return pltpu.make_async_remote_copy(
                src_ref=out_ref.at[rows(oz, k), :],
                dst_ref=out_ref.at[rows(oz, k), :],
                send_sem=rzx_send.at[k - RX_LO],
                recv_sem=xr_sems.at[k - RX_LO],
                device_id=xpeer,
                device_id_type=pl.DeviceIdType.MESH,
            )

        def xr(k):
            return pltpu.make_async_remote_copy(
                src_ref=out_ref.at[rows(od, k), :],
                dst_ref=out_ref.at[rows(od, k), :],
                send_sem=rzx_send.at[k - RX_LO],
                recv_sem=xr_sems.at[k - RX_LO],
                device_id=xpeer,
                device_id_type=pl.DeviceIdType.MESH,
            )

        def zr(k):
            return pltpu.make_async_remote_copy(
                src_ref=out_ref.at[rows(od, k), :],
                dst_ref=out_ref.at[rows(od, k), :],
                send_sem=rxz_send.at[k - RZ_LO],
                recv_sem=zr_sems.at[k - RZ_LO],
                device_id=zpart,
                device_id_type=pl.DeviceIdType.MESH,
            )

        bar = pltpu.get_barrier_semaphore()
        for nbr in (ypeer, xpeer, zpart):
            pl.semaphore_signal(
                bar, inc=1, device_id=nbr,
                device_id_type=pl.DeviceIdType.MESH,
            )
        pl.semaphore_wait(bar, 3)

        for r in range(NY + 8):
            if r < NY:
                if r >= S_SND:
                    y_rdma(r - S_SND).wait_send()
                y_rdma(r).start()
                if r >= S_L:
                    if r - S_L < NC:
                        sum_x(r - S_L).wait_send()
                        sum_z(r - S_L).wait_send()
                    out_copy(r - S_L).wait()
                xload(r).start()
            s = r - 2
            if 0 <= s < NY:
                xload(s).wait()
                y_rdma(s).wait_recv()
                xloc[s % S_L] = xloc[s % S_L] + recv_y[s % S_Y]
                out_copy(s).start()
                if s < NC:
                    sum_x(s).start()
                    sum_z(s).start()
            t = r - 3
            if 0 <= t < NC:
                xin(t).wait_recv()
                if RZ_LO <= t < RZ_HI:
                    relay_xz(t).start()
                zin(t).wait_recv()
                if RX_LO <= t < RX_HI:
                    relay_zx(t).start()
            u = r - 5
            if RX_LO <= u < RX_HI:
                xr(u).wait_recv()
            if RZ_LO <= u < RZ_HI:
                zr(u).wait_recv()

        for k in range(NY - S_SND, NY):
            y_rdma(k).wait_send()
        for k in range(NY - S_L, NY):
            out_copy(k).wait()
        for k in range(max(0, NY - S_L), NC):
            sum_x(k).wait_send()
            sum_z(k).wait_send()
        for k in range(RZ_LO, RZ_HI):
            relay_xz(k).wait_send()
        for k in range(RX_LO, RX_HI):
            relay_zx(k).wait_send()

    return pl.pallas_call(
        body,
        in_specs=[pl.BlockSpec(memory_space=pltpu.MemorySpace.HBM)],
        out_specs=pl.BlockSpec(memory_space=pltpu.MemorySpace.HBM),
        out_shape=jax.ShapeDtypeStruct((m, n), x.dtype),
        scratch_shapes=[
            pltpu.VMEM((S_Y, c_m, n), x.dtype),
            pltpu.VMEM((S_L, c_m, n), x.dtype),
            pltpu.SemaphoreType.DMA((S_SND,)),
            pltpu.SemaphoreType.DMA((S_Y,)),
            pltpu.SemaphoreType.DMA((S_L,)),
            pltpu.SemaphoreType.DMA((S_SND,)),
            pltpu.SemaphoreType.DMA((S_SND,)),
            pltpu.SemaphoreType.DMA((S_SND,)),
            pltpu.SemaphoreType.DMA((RZ_HI - RZ_LO,)),
            pltpu.SemaphoreType.DMA((RX_HI - RX_LO,)),
            pltpu.SemaphoreType.DMA((NC,)),
            pltpu.SemaphoreType.DMA((NC,)),
            pltpu.SemaphoreType.DMA((RX_HI - RX_LO,)),
            pltpu.SemaphoreType.DMA((RZ_HI - RZ_LO,)),
        ],
        compiler_params=pltpu.CompilerParams(
            collective_id=0,
            vmem_limit_bytes=60 * 1024 * 1024,
        ),
    )(x)


# device time: 660204 ns/iter; 1.0420x vs baseline; 1.0420x over previous
import jax
import jax.numpy as jnp
from jax import lax
from jax.experimental import pallas as pl
from jax.experimental.pallas import tpu as pltpu

NC = 16
S_Y = 6
S_L = 4
S_SND = 4


def kernel(x):
    m, n = x.shape
    q_m = m // 4
    c_m = q_m // NC

    def body(x_ref, out_ref, recv_y, xloc,
             y_send, y_recv, load_sems, oc_sems, sx_send, sz_send,
             rxz_send, rzx_send, xin_sems, zin_sems, xr_sems, zr_sems):
        my_x = lax.axis_index("x")
        my_y = lax.axis_index("y")
        my_z = lax.axis_index("z")
        zbit = lax.rem(my_z, 2)
        ypeer = (my_x, 1 - my_y, my_z)
        xpeer = (1 - my_x, my_y, my_z)
        zpart = (my_x, my_y, my_z + 1 - 2 * zbit)

        o = 2 * my_x + zbit
        ox = lax.rem(o + 2, 4)
        oz = o + 1 - 2 * lax.rem(o, 2)
        od = lax.rem(oz + 2, 4)

        def rows(quarter, k):
            return pl.ds(quarter * q_m + k * c_m, c_m)

        def y_rdma(k):
            return pltpu.make_async_remote_copy(
                src_ref=x_ref.at[rows(o, k), :],
                dst_ref=recv_y.at[k % S_Y],
                send_sem=y_send.at[k % S_SND],
                recv_sem=y_recv.at[k % S_Y],
                device_id=ypeer,
                device_id_type=pl.DeviceIdType.MESH,
            )

        def xload(k):
            return pltpu.make_async_copy(
                x_ref.at[rows(o, k), :],
                xloc.at[k % S_L],
                load_sems.at[k % S_L],
            )

        def out_copy(k):
            return pltpu.make_async_copy(
                xloc.at[k % S_L],
                out_ref.at[rows(o, k), :],
                oc_sems.at[k % S_SND],
            )

        def sum_x(k):
            return pltpu.make_async_remote_copy(
                src_ref=xloc.at[k % S_L],
                dst_ref=out_ref.at[rows(o, k), :],
                send_sem=sx_send.at[k % S_SND],
                recv_sem=xin_sems.at[k],
                device_id=xpeer,
                device_id_type=pl.DeviceIdType.MESH,
            )

        def sum_z(k):
            return pltpu.make_async_remote_copy(
                src_ref=xloc.at[k % S_L],
                dst_ref=out_ref.at[rows(o, k), :],
                send_sem=sz_send.at[k % S_SND],
                recv_sem=zin_sems.at[k],
                device_id=zpart,
                device_id_type=pl.DeviceIdType.MESH,
            )

        def xin(k):
            return pltpu.make_async_remote_copy(
                src_ref=xloc.at[0],
                dst_ref=out_ref.at[rows(ox, k), :],
                send_sem=sx_send.at[0],
                recv_sem=xin_sems.at[k],
                device_id=xpeer,
                device_id_type=pl.DeviceIdType.MESH,
            )

        def zin(k):
            return pltpu.make_async_remote_copy(
                src_ref=xloc.at[0],
                dst_ref=out_ref.at[rows(oz, k), :],
                send_sem=sz_send.at[0],
                recv_sem=zin_sems.at[k],
                device_id=zpart,
                device_id_type=pl.DeviceIdType.MESH,
            )

        def relay_xz(k):
            return pltpu.make_async_remote_copy(
                src_ref=out_ref.at[rows(ox, k), :],
                dst_ref=out_ref.at[rows(ox, k), :],
                send_sem=rxz_send.at[k // 2],
                recv_sem=zr_sems.at[k // 2],
                device_id=zpart,
                device_id_type=pl.DeviceIdType.MESH,
            )

        def relay_zx(k):
            return pltpu.make_async_remote_copy(
                src_ref=out_ref.at[rows(oz, k), :],
                dst_ref=out_ref.at[rows(oz, k), :],
                send_sem=rzx_send.at[k // 2],
                recv_sem=xr_sems.at[k // 2],
                device_id=xpeer,
                device_id_type=pl.DeviceIdType.MESH,
            )

        def xr(k):
            return pltpu.make_async_remote_copy(
                src_ref=out_ref.at[rows(od, k), :],
                dst_ref=out_ref.at[rows(od, k), :],
                send_sem=rzx_send.at[k // 2],
                recv_sem=xr_sems.at[k // 2],
                device_id=xpeer,
                device_id_type=pl.DeviceIdType.MESH,
            )

        def zr(k):
            return pltpu.make_async_remote_copy(
                src_ref=out_ref.at[rows(od, k), :],
                dst_ref=out_ref.at[rows(od, k), :],
                send_sem=rxz_send.at[k // 2],
                recv_sem=zr_sems.at[k // 2],
                device_id=zpart,
                device_id_type=pl.DeviceIdType.MESH,
            )

        bar = pltpu.get_barrier_semaphore()
        for nbr in (ypeer, xpeer, zpart):
            pl.semaphore_signal(
                bar, inc=1, device_id=nbr,
                device_id_type=pl.DeviceIdType.MESH,
            )
        pl.semaphore_wait(bar, 3)

        for r in range(NC + 8):
            if r < NC:
                if r >= S_SND:
                    y_rdma(r - S_SND).wait_send()
                y_rdma(r).start()
                if r >= S_L:
                    sum_x(r - S_L).wait_send()
                    sum_z(r - S_L).wait_send()
                    out_copy(r - S_L).wait()
                xload(r).start()
            s = r - 2
            if 0 <= s < NC:
                xload(s).wait()
                y_rdma(s).wait_recv()
                xloc[s % S_L] = xloc[s % S_L] + recv_y[s % S_Y]
                out_copy(s).start()
                sum_x(s).start()
                sum_z(s).start()
            t = r - 3
            if 0 <= t < NC:
                xin(t).wait_recv()
                if t % 2 == 0:
                    relay_xz(t).start()
                zin(t).wait_recv()
                if t % 2 == 1:
                    relay_zx(t).start()
            u = r - 5
            if 0 <= u < NC:
                if u % 2 == 1:
                    xr(u).wait_recv()
                else:
                    zr(u).wait_recv()

        for k in range(NC - S_SND, NC):
            y_rdma(k).wait_send()
        for k in range(NC - S_L, NC):
            sum_x(k).wait_send()
            sum_z(k).wait_send()
            out_copy(k).wait()
        for k in range(0, NC, 2):
            relay_xz(k).wait_send()
        for k in range(1, NC, 2):
            relay_zx(k).wait_send()

    return pl.pallas_call(
        body,
        in_specs=[pl.BlockSpec(memory_space=pltpu.MemorySpace.HBM)],
        out_specs=pl.BlockSpec(memory_space=pltpu.MemorySpace.HBM),
        out_shape=jax.ShapeDtypeStruct((m, n), x.dtype),
        scratch_shapes=[
            pltpu.VMEM((S_Y, c_m, n), x.dtype),
            pltpu.VMEM((S_L, c_m, n), x.dtype),
            pltpu.SemaphoreType.DMA((S_SND,)),
            pltpu.SemaphoreType.DMA((S_Y,)),
            pltpu.SemaphoreType.DMA((S_L,)),
            pltpu.SemaphoreType.DMA((S_SND,)),
            pltpu.SemaphoreType.DMA((S_SND,)),
            pltpu.SemaphoreType.DMA((S_SND,)),
            pltpu.SemaphoreType.DMA((NC // 2,)),
            pltpu.SemaphoreType.DMA((NC // 2,)),
            pltpu.SemaphoreType.DMA((NC,)),
            pltpu.SemaphoreType.DMA((NC,)),
            pltpu.SemaphoreType.DMA((NC // 2,)),
            pltpu.SemaphoreType.DMA((NC // 2,)),
        ],
        compiler_params=pltpu.CompilerParams(
            collective_id=0,
            vmem_limit_bytes=60 * 1024 * 1024,
        ),
    )(x)
